# baseline (speedup 1.0000x reference)
import functools

import jax
import jax.numpy as jnp
from jax import lax
from jax.experimental import pallas as pl
from jax.experimental.pallas import tpu as pltpu
from jax.experimental.pallas import tpu_sc as plsc

N = 10000
D = 128
NC = 2
NS = 16
NW = NC * NS
CHUNK = 128
KB = 8
GB = 10
CHUNKS = GB * KB
NE = 320000
EPAD = NW * CHUNKS * CHUNK - NE
NPAD = 10112
RPT = NPAD // NS

DEGW = 128
BLK = 1000
NBLK = N // BLK

def _mesh():
    return plsc.VectorSubcoreMesh(core_axis_name="c", subcore_axis_name="s")


def _zero_rows(zbuf, nrows, width):
    @pl.loop(0, nrows)
    def _(i):
        @pl.loop(0, width, step=16)
        def _(j):
            zbuf[i, pl.ds(j, 16)] = jnp.zeros((16,), jnp.float32)


def _zero_acc_slice(acc_sh, zbuf, sid):
    base = sid * RPT

    @pl.loop(0, RPT, step=8)
    def _(r):
        pltpu.sync_copy(zbuf, acc_sh.at[pl.ds(base + r, 8)])


def _sc_degree(dst_idx):
    @functools.partial(
        pl.kernel,
        out_type=jax.ShapeDtypeStruct((NC, NPAD, DEGW), jnp.float32),
        mesh=_mesh(),
        scratch_types=[
            pltpu.VMEM_SHARED((NPAD, DEGW), jnp.float32),
            pltpu.VMEM((KB, CHUNK), jnp.int32),
            pltpu.VMEM((CHUNK, DEGW), jnp.float32),
            pltpu.VMEM((8, DEGW), jnp.float32),
        ],
    )
    def k(dst_hbm, out_hbm, acc_sh, dstv, ones_v, zbuf):
        cid = lax.axis_index("c")
        sid = lax.axis_index("s")
        wid = sid * NC + cid
        _zero_rows(zbuf, 8, DEGW)

        @pl.loop(0, CHUNK)
        def _(i):
            @pl.loop(0, DEGW, step=16)
            def _(j):
                ones_v[i, pl.ds(j, 16)] = jnp.ones((16,), jnp.float32)

        _zero_acc_slice(acc_sh, zbuf, sid)
        plsc.subcore_barrier()

        @pl.loop(0, GB)
        def _(g):
            pltpu.sync_copy(dst_hbm.at[wid, g], dstv)

            @pl.loop(0, KB)
            def _(c):
                pltpu.sync_copy(ones_v, acc_sh.at[dstv.at[c]], add=True)

        plsc.subcore_barrier()
        pltpu.sync_copy(
            acc_sh.at[pl.ds(sid * RPT, RPT)],
            out_hbm.at[cid, pl.ds(sid * RPT, RPT)],
        )

    return k(dst_idx)


def _sc_gather_add(h, src_idx, dst_idx):
    @functools.partial(
        pl.kernel,
        out_type=jax.ShapeDtypeStruct((NC, NPAD, D), jnp.float32),
        mesh=_mesh(),
        scratch_types=[
            pltpu.VMEM_SHARED((NPAD, D), jnp.float32),
            pltpu.VMEM((2, KB, CHUNK), jnp.int32),
            pltpu.VMEM((2, KB, CHUNK), jnp.int32),
            pltpu.VMEM((CHUNK, D), jnp.float32),
            pltpu.VMEM((CHUNK, D), jnp.float32),
            pltpu.VMEM((8, D), jnp.float32),
            pltpu.SemaphoreType.DMA,
            pltpu.SemaphoreType.DMA,
        ],
    )
    def k(h_hbm, src_hbm, dst_hbm, out_hbm, acc_sh, srcv2, dstv2, rows_a,
          rows_b, zbuf, sem_a, sem_b):
        cid = lax.axis_index("c")
        sid = lax.axis_index("s")
        wid = sid * NC + cid
        _zero_rows(zbuf, 8, D)

        _zero_acc_slice(acc_sh, zbuf, sid)
        pltpu.sync_copy(src_hbm.at[wid, 0], srcv2.at[0])
        pltpu.sync_copy(dst_hbm.at[wid, 0], dstv2.at[0])
        plsc.subcore_barrier()

        @pl.loop(0, GB)
        def _(g):
            par = g % 2

            @pl.loop(0, KB)
            def _(c):
                pltpu.async_copy(
                    h_hbm.at[srcv2.at[par].at[c]], rows_a, sem_a
                ).wait()
                pltpu.sync_copy(rows_a, acc_sh.at[dstv2.at[par].at[c]], add=True)

            @pl.when(g + 1 < GB)
            def _():
                pltpu.sync_copy(src_hbm.at[wid, g + 1], srcv2.at[(g + 1) % 2])
                pltpu.sync_copy(dst_hbm.at[wid, g + 1], dstv2.at[(g + 1) % 2])

        plsc.subcore_barrier()
        pltpu.sync_copy(
            acc_sh.at[pl.ds(sid * RPT, RPT)],
            out_hbm.at[cid, pl.ds(sid * RPT, RPT)],
        )

    return k(h, src_idx, dst_idx)


def _tc_linear(x, W, b):
    def body(x_ref, w_ref, b_ref, o_ref):
        o_ref[...] = (
            jnp.dot(x_ref[...], w_ref[...], preferred_element_type=jnp.float32)
            + b_ref[...]
        )

    return pl.pallas_call(
        body,
        grid=(NBLK,),
        in_specs=[
            pl.BlockSpec((BLK, D), lambda i: (i, 0)),
            pl.BlockSpec((D, D), lambda i: (0, 0)),
            pl.BlockSpec((1, D), lambda i: (0, 0)),
        ],
        out_specs=pl.BlockSpec((BLK, D), lambda i: (i, 0)),
        out_shape=jax.ShapeDtypeStruct((N, D), jnp.float32),
    )(x, W, b.reshape(1, D))


def _dinv_block(deg_ref):
    d = deg_ref[0, :, 0:1] + deg_ref[1, :, 0:1]
    return lax.rsqrt(jnp.maximum(d, 1.0))


def _tc_prescale(h, degp):
    def body(h_ref, deg_ref, o_ref):
        o_ref[...] = h_ref[...] * _dinv_block(deg_ref)

    return pl.pallas_call(
        body,
        grid=(NBLK,),
        in_specs=[
            pl.BlockSpec((BLK, D), lambda i: (i, 0)),
            pl.BlockSpec((NC, BLK, DEGW), lambda i: (0, i, 0)),
        ],
        out_specs=pl.BlockSpec((BLK, D), lambda i: (i, 0)),
        out_shape=jax.ShapeDtypeStruct((N, D), jnp.float32),
    )(h, degp)


def _tc_mid(aggp, degp, W, b):
    def body(agg_ref, deg_ref, w_ref, b_ref, o_ref):
        dinv = _dinv_block(deg_ref)
        x1 = jnp.maximum((agg_ref[0] + agg_ref[1]) * dinv, 0.0)
        h2 = jnp.dot(x1, w_ref[...], preferred_element_type=jnp.float32) + b_ref[...]
        o_ref[...] = h2 * dinv

    return pl.pallas_call(
        body,
        grid=(NBLK,),
        in_specs=[
            pl.BlockSpec((NC, BLK, D), lambda i: (0, i, 0)),
            pl.BlockSpec((NC, BLK, DEGW), lambda i: (0, i, 0)),
            pl.BlockSpec((D, D), lambda i: (0, 0)),
            pl.BlockSpec((1, D), lambda i: (0, 0)),
        ],
        out_specs=pl.BlockSpec((BLK, D), lambda i: (i, 0)),
        out_shape=jax.ShapeDtypeStruct((N, D), jnp.float32),
    )(aggp, degp, W, b.reshape(1, D))


def _tc_logsoftmax(aggp, degp):
    def body(agg_ref, deg_ref, o_ref):
        a = (agg_ref[0] + agg_ref[1]) * _dinv_block(deg_ref)
        m = jnp.max(a, axis=1, keepdims=True)
        ex = jnp.exp(a - m)
        lse = jnp.log(jnp.sum(ex, axis=1, keepdims=True)) + m
        o_ref[...] = a - lse

    return pl.pallas_call(
        body,
        grid=(NBLK,),
        in_specs=[
            pl.BlockSpec((NC, BLK, D), lambda i: (0, i, 0)),
            pl.BlockSpec((NC, BLK, DEGW), lambda i: (0, i, 0)),
        ],
        out_specs=pl.BlockSpec((BLK, D), lambda i: (i, 0)),
        out_shape=jax.ShapeDtypeStruct((N, D), jnp.float32),
    )(aggp, degp)


def kernel(input_feature, edge_index, W1, b1, W2, b2):
    src = jnp.concatenate(
        [edge_index[0], jnp.zeros((EPAD,), jnp.int32)]
    ).reshape(NW, GB, KB, CHUNK)
    dst = jnp.concatenate(
        [edge_index[1], jnp.full((EPAD,), N, jnp.int32)]
    ).reshape(NW, GB, KB, CHUNK)

    degp = _sc_degree(dst)
    h1 = _tc_linear(input_feature, W1, b1)
    h1p = _tc_prescale(h1, degp)
    agg1 = _sc_gather_add(h1p, src, dst)
    h2p = _tc_mid(agg1, degp, W2, b2)
    agg2 = _sc_gather_add(h2p, src, dst)
    return _tc_logsoftmax(agg2, degp)


# speedup vs baseline: 1.0310x; 1.0310x over previous
import functools

import jax
import jax.numpy as jnp
from jax import lax
from jax.experimental import pallas as pl
from jax.experimental.pallas import tpu as pltpu
from jax.experimental.pallas import tpu_sc as plsc

N = 10000
D = 128
NC = 2
NS = 16
NW = NC * NS
CHUNK = 128
KB = 8
GB = 10
CHUNKS = GB * KB
NE = 320000
EPAD = NW * CHUNKS * CHUNK - NE
NPAD = 10112
RPT = NPAD // NS

DEGW = 128
BLK = 1000
NBLK = N // BLK

def _mesh():
    return plsc.VectorSubcoreMesh(core_axis_name="c", subcore_axis_name="s")


def _zero_rows(zbuf, nrows, width):
    @pl.loop(0, nrows)
    def _(i):
        @pl.loop(0, width, step=16)
        def _(j):
            zbuf[i, pl.ds(j, 16)] = jnp.zeros((16,), jnp.float32)


def _zero_acc_slice(acc_sh, zbuf, sid):
    base = sid * RPT

    @pl.loop(0, RPT, step=8)
    def _(r):
        pltpu.sync_copy(zbuf, acc_sh.at[pl.ds(base + r, 8)])


def _sc_degree(dst_idx):
    @functools.partial(
        pl.kernel,
        out_type=jax.ShapeDtypeStruct((NC, NPAD, DEGW), jnp.float32),
        mesh=_mesh(),
        scratch_types=[
            pltpu.VMEM_SHARED((NPAD, DEGW), jnp.float32),
            pltpu.VMEM((CHUNKS, CHUNK), jnp.int32),
            pltpu.VMEM((CHUNK, DEGW), jnp.float32),
            pltpu.VMEM((8, DEGW), jnp.float32),
        ],
    )
    def k(dst_hbm, out_hbm, acc_sh, dstv, ones_v, zbuf):
        cid = lax.axis_index("c")
        sid = lax.axis_index("s")
        wid = sid * NC + cid
        _zero_rows(zbuf, 8, DEGW)

        @pl.loop(0, CHUNK)
        def _(i):
            @pl.loop(0, DEGW, step=16)
            def _(j):
                ones_v[i, pl.ds(j, 16)] = jnp.ones((16,), jnp.float32)

        _zero_acc_slice(acc_sh, zbuf, sid)
        pltpu.sync_copy(dst_hbm.at[wid], dstv)
        plsc.subcore_barrier()

        @pl.loop(0, CHUNKS)
        def _(j):
            pltpu.sync_copy(ones_v, acc_sh.at[dstv.at[j]], add=True)

        plsc.subcore_barrier()
        pltpu.sync_copy(
            acc_sh.at[pl.ds(sid * RPT, RPT)],
            out_hbm.at[cid, pl.ds(sid * RPT, RPT)],
        )

    return k(dst_idx)


def _sc_gather_add(h, src_idx, dst_idx):
    @functools.partial(
        pl.kernel,
        out_type=jax.ShapeDtypeStruct((NC, NPAD, D), jnp.float32),
        mesh=_mesh(),
        scratch_types=[
            pltpu.VMEM_SHARED((NPAD, D), jnp.float32),
            pltpu.VMEM((CHUNKS, CHUNK), jnp.int32),
            pltpu.VMEM((CHUNKS, CHUNK), jnp.int32),
            pltpu.VMEM((CHUNK, D), jnp.float32),
            pltpu.VMEM((8, D), jnp.float32),
            pltpu.SemaphoreType.DMA,
        ],
    )
    def k(h_hbm, src_hbm, dst_hbm, out_hbm, acc_sh, srcv, dstv, rows_a,
          zbuf, sem_a):
        cid = lax.axis_index("c")
        sid = lax.axis_index("s")
        wid = sid * NC + cid
        _zero_rows(zbuf, 8, D)

        _zero_acc_slice(acc_sh, zbuf, sid)
        pltpu.sync_copy(src_hbm.at[wid], srcv)
        pltpu.sync_copy(dst_hbm.at[wid], dstv)
        plsc.subcore_barrier()

        @pl.loop(0, CHUNKS)
        def _(j):
            pltpu.async_copy(h_hbm.at[srcv.at[j]], rows_a, sem_a).wait()
            pltpu.sync_copy(rows_a, acc_sh.at[dstv.at[j]], add=True)

        plsc.subcore_barrier()
        pltpu.sync_copy(
            acc_sh.at[pl.ds(sid * RPT, RPT)],
            out_hbm.at[cid, pl.ds(sid * RPT, RPT)],
        )

    return k(h, src_idx, dst_idx)


def _tc_linear(x, W, b):
    def body(x_ref, w_ref, b_ref, o_ref):
        o_ref[...] = (
            jnp.dot(x_ref[...], w_ref[...], preferred_element_type=jnp.float32)
            + b_ref[...]
        )

    return pl.pallas_call(
        body,
        grid=(NBLK,),
        in_specs=[
            pl.BlockSpec((BLK, D), lambda i: (i, 0)),
            pl.BlockSpec((D, D), lambda i: (0, 0)),
            pl.BlockSpec((1, D), lambda i: (0, 0)),
        ],
        out_specs=pl.BlockSpec((BLK, D), lambda i: (i, 0)),
        out_shape=jax.ShapeDtypeStruct((N, D), jnp.float32),
    )(x, W, b.reshape(1, D))


def _dinv_block(deg_ref):
    d = deg_ref[0, :, 0:1] + deg_ref[1, :, 0:1]
    return lax.rsqrt(jnp.maximum(d, 1.0))


def _tc_prescale(h, degp):
    def body(h_ref, deg_ref, o_ref):
        o_ref[...] = h_ref[...] * _dinv_block(deg_ref)

    return pl.pallas_call(
        body,
        grid=(NBLK,),
        in_specs=[
            pl.BlockSpec((BLK, D), lambda i: (i, 0)),
            pl.BlockSpec((NC, BLK, DEGW), lambda i: (0, i, 0)),
        ],
        out_specs=pl.BlockSpec((BLK, D), lambda i: (i, 0)),
        out_shape=jax.ShapeDtypeStruct((N, D), jnp.float32),
    )(h, degp)


def _tc_mid(aggp, degp, W, b):
    def body(agg_ref, deg_ref, w_ref, b_ref, o_ref):
        dinv = _dinv_block(deg_ref)
        x1 = jnp.maximum((agg_ref[0] + agg_ref[1]) * dinv, 0.0)
        h2 = jnp.dot(x1, w_ref[...], preferred_element_type=jnp.float32) + b_ref[...]
        o_ref[...] = h2 * dinv

    return pl.pallas_call(
        body,
        grid=(NBLK,),
        in_specs=[
            pl.BlockSpec((NC, BLK, D), lambda i: (0, i, 0)),
            pl.BlockSpec((NC, BLK, DEGW), lambda i: (0, i, 0)),
            pl.BlockSpec((D, D), lambda i: (0, 0)),
            pl.BlockSpec((1, D), lambda i: (0, 0)),
        ],
        out_specs=pl.BlockSpec((BLK, D), lambda i: (i, 0)),
        out_shape=jax.ShapeDtypeStruct((N, D), jnp.float32),
    )(aggp, degp, W, b.reshape(1, D))


def _tc_logsoftmax(aggp, degp):
    def body(agg_ref, deg_ref, o_ref):
        a = (agg_ref[0] + agg_ref[1]) * _dinv_block(deg_ref)
        m = jnp.max(a, axis=1, keepdims=True)
        ex = jnp.exp(a - m)
        lse = jnp.log(jnp.sum(ex, axis=1, keepdims=True)) + m
        o_ref[...] = a - lse

    return pl.pallas_call(
        body,
        grid=(NBLK,),
        in_specs=[
            pl.BlockSpec((NC, BLK, D), lambda i: (0, i, 0)),
            pl.BlockSpec((NC, BLK, DEGW), lambda i: (0, i, 0)),
        ],
        out_specs=pl.BlockSpec((BLK, D), lambda i: (i, 0)),
        out_shape=jax.ShapeDtypeStruct((N, D), jnp.float32),
    )(aggp, degp)


def kernel(input_feature, edge_index, W1, b1, W2, b2):
    src = jnp.concatenate(
        [edge_index[0], jnp.zeros((EPAD,), jnp.int32)]
    ).reshape(NW, CHUNKS, CHUNK)
    dst = jnp.concatenate(
        [edge_index[1], jnp.full((EPAD,), N, jnp.int32)]
    ).reshape(NW, CHUNKS, CHUNK)

    degp = _sc_degree(dst)
    h1 = _tc_linear(input_feature, W1, b1)
    h1p = _tc_prescale(h1, degp)
    agg1 = _sc_gather_add(h1p, src, dst)
    h2p = _tc_mid(agg1, degp, W2, b2)
    agg2 = _sc_gather_add(h2p, src, dst)
    return _tc_logsoftmax(agg2, degp)


# speedup vs baseline: 1.7997x; 1.7456x over previous
import functools

import jax
import jax.numpy as jnp
from jax import lax
from jax.experimental import pallas as pl
from jax.experimental.pallas import tpu as pltpu
from jax.experimental.pallas import tpu_sc as plsc

N = 10000
D = 128
NC = 2
NS = 16
NW = NC * NS
CHUNK = 120
KB = 6
GB = 14
CHUNKS = GB * KB
NE = 320000
EPAD = NW * CHUNKS * CHUNK - NE
NPAD = 10112
RPT = NPAD // NS

DEGW = 128
BLK = 1000
NBLK = N // BLK

def _mesh():
    return plsc.VectorSubcoreMesh(core_axis_name="c", subcore_axis_name="s")


def _zero_rows(zbuf, nrows, width):
    @pl.loop(0, nrows)
    def _(i):
        @pl.loop(0, width, step=16)
        def _(j):
            zbuf[i, pl.ds(j, 16)] = jnp.zeros((16,), jnp.float32)


def _zero_acc_slice(acc_sh, zbuf, sid):
    base = sid * RPT

    @pl.loop(0, RPT, step=8)
    def _(r):
        pltpu.sync_copy(zbuf, acc_sh.at[pl.ds(base + r, 8)])


def _sc_degree(dst_idx):
    @functools.partial(
        pl.kernel,
        out_type=jax.ShapeDtypeStruct((NC, NPAD, DEGW), jnp.float32),
        mesh=_mesh(),
        scratch_types=[
            pltpu.VMEM_SHARED((NPAD, DEGW), jnp.float32),
            pltpu.VMEM((KB, CHUNK), jnp.int32),
            pltpu.VMEM((CHUNK, DEGW), jnp.float32),
            pltpu.VMEM((8, DEGW), jnp.float32),
        ],
    )
    def k(dst_hbm, out_hbm, acc_sh, dstv, ones_v, zbuf):
        cid = lax.axis_index("c")
        sid = lax.axis_index("s")
        wid = sid * NC + cid
        _zero_rows(zbuf, 8, DEGW)

        @pl.loop(0, CHUNK)
        def _(i):
            @pl.loop(0, DEGW, step=16)
            def _(j):
                ones_v[i, pl.ds(j, 16)] = jnp.ones((16,), jnp.float32)

        _zero_acc_slice(acc_sh, zbuf, sid)
        plsc.subcore_barrier()

        @pl.loop(0, GB)
        def _(g):
            pltpu.sync_copy(dst_hbm.at[wid, g], dstv)

            @pl.loop(0, KB)
            def _(c):
                pltpu.sync_copy(ones_v, acc_sh.at[dstv.at[c]], add=True)

        plsc.subcore_barrier()
        pltpu.sync_copy(
            acc_sh.at[pl.ds(sid * RPT, RPT)],
            out_hbm.at[cid, pl.ds(sid * RPT, RPT)],
        )

    return k(dst_idx)


def _sc_gather_add(h, src_idx, dst_idx):
    @functools.partial(
        pl.kernel,
        out_type=jax.ShapeDtypeStruct((NC, NPAD, D), jnp.float32),
        mesh=_mesh(),
        scratch_types=[
            pltpu.VMEM_SHARED((NPAD, D), jnp.float32),
            pltpu.VMEM((KB, CHUNK), jnp.int32),
            pltpu.VMEM((KB, CHUNK), jnp.int32),
            pltpu.VMEM((CHUNK, D), jnp.float32),
            pltpu.VMEM((CHUNK, D), jnp.float32),
            pltpu.VMEM((8, D), jnp.float32),
            pltpu.SemaphoreType.DMA,
            pltpu.SemaphoreType.DMA,
        ],
    )
    def k(h_hbm, src_hbm, dst_hbm, out_hbm, acc_sh, srcv2, dstv2, rows_a,
          rows_b, zbuf, sem_a, sem_b):
        cid = lax.axis_index("c")
        sid = lax.axis_index("s")
        wid = sid * NC + cid
        _zero_rows(zbuf, 8, D)

        _zero_acc_slice(acc_sh, zbuf, sid)
        pltpu.sync_copy(src_hbm.at[wid, 0], srcv2)
        pltpu.sync_copy(dst_hbm.at[wid, 0], dstv2)
        plsc.subcore_barrier()

        pltpu.async_copy(h_hbm.at[srcv2.at[0]], rows_a, sem_a)

        @pl.loop(0, GB)
        def _(g):
            @pl.loop(0, KB, step=2)
            def _(c):
                pltpu.make_async_copy(
                    h_hbm.at[pl.ds(0, CHUNK)], rows_a, sem_a
                ).wait()
                pltpu.async_copy(h_hbm.at[srcv2.at[c + 1]], rows_b, sem_b)
                pltpu.sync_copy(rows_a, acc_sh.at[dstv2.at[c]], add=True)
                pltpu.make_async_copy(
                    h_hbm.at[pl.ds(0, CHUNK)], rows_b, sem_b
                ).wait()

                @pl.when(c + 2 < KB)
                def _():
                    pltpu.async_copy(h_hbm.at[srcv2.at[c + 2]], rows_a, sem_a)

                pltpu.sync_copy(rows_b, acc_sh.at[dstv2.at[c + 1]], add=True)

            @pl.when(g + 1 < GB)
            def _():
                pltpu.sync_copy(src_hbm.at[wid, g + 1], srcv2)
                pltpu.sync_copy(dst_hbm.at[wid, g + 1], dstv2)
                pltpu.async_copy(h_hbm.at[srcv2.at[0]], rows_a, sem_a)

        plsc.subcore_barrier()
        pltpu.sync_copy(
            acc_sh.at[pl.ds(sid * RPT, RPT)],
            out_hbm.at[cid, pl.ds(sid * RPT, RPT)],
        )

    return k(h, src_idx, dst_idx)


def _tc_linear(x, W, b):
    def body(x_ref, w_ref, b_ref, o_ref):
        o_ref[...] = (
            jnp.dot(x_ref[...], w_ref[...], preferred_element_type=jnp.float32)
            + b_ref[...]
        )

    return pl.pallas_call(
        body,
        grid=(NBLK,),
        in_specs=[
            pl.BlockSpec((BLK, D), lambda i: (i, 0)),
            pl.BlockSpec((D, D), lambda i: (0, 0)),
            pl.BlockSpec((1, D), lambda i: (0, 0)),
        ],
        out_specs=pl.BlockSpec((BLK, D), lambda i: (i, 0)),
        out_shape=jax.ShapeDtypeStruct((N, D), jnp.float32),
    )(x, W, b.reshape(1, D))


def _dinv_block(deg_ref):
    d = deg_ref[0, :, 0:1] + deg_ref[1, :, 0:1]
    return lax.rsqrt(jnp.maximum(d, 1.0))


def _tc_prescale(h, degp):
    def body(h_ref, deg_ref, o_ref):
        o_ref[...] = h_ref[...] * _dinv_block(deg_ref)

    return pl.pallas_call(
        body,
        grid=(NBLK,),
        in_specs=[
            pl.BlockSpec((BLK, D), lambda i: (i, 0)),
            pl.BlockSpec((NC, BLK, DEGW), lambda i: (0, i, 0)),
        ],
        out_specs=pl.BlockSpec((BLK, D), lambda i: (i, 0)),
        out_shape=jax.ShapeDtypeStruct((N, D), jnp.float32),
    )(h, degp)


def _tc_mid(aggp, degp, W, b):
    def body(agg_ref, deg_ref, w_ref, b_ref, o_ref):
        dinv = _dinv_block(deg_ref)
        x1 = jnp.maximum((agg_ref[0] + agg_ref[1]) * dinv, 0.0)
        h2 = jnp.dot(x1, w_ref[...], preferred_element_type=jnp.float32) + b_ref[...]
        o_ref[...] = h2 * dinv

    return pl.pallas_call(
        body,
        grid=(NBLK,),
        in_specs=[
            pl.BlockSpec((NC, BLK, D), lambda i: (0, i, 0)),
            pl.BlockSpec((NC, BLK, DEGW), lambda i: (0, i, 0)),
            pl.BlockSpec((D, D), lambda i: (0, 0)),
            pl.BlockSpec((1, D), lambda i: (0, 0)),
        ],
        out_specs=pl.BlockSpec((BLK, D), lambda i: (i, 0)),
        out_shape=jax.ShapeDtypeStruct((N, D), jnp.float32),
    )(aggp, degp, W, b.reshape(1, D))


def _tc_logsoftmax(aggp, degp):
    def body(agg_ref, deg_ref, o_ref):
        a = (agg_ref[0] + agg_ref[1]) * _dinv_block(deg_ref)
        m = jnp.max(a, axis=1, keepdims=True)
        ex = jnp.exp(a - m)
        lse = jnp.log(jnp.sum(ex, axis=1, keepdims=True)) + m
        o_ref[...] = a - lse

    return pl.pallas_call(
        body,
        grid=(NBLK,),
        in_specs=[
            pl.BlockSpec((NC, BLK, D), lambda i: (0, i, 0)),
            pl.BlockSpec((NC, BLK, DEGW), lambda i: (0, i, 0)),
        ],
        out_specs=pl.BlockSpec((BLK, D), lambda i: (i, 0)),
        out_shape=jax.ShapeDtypeStruct((N, D), jnp.float32),
    )(aggp, degp)


def kernel(input_feature, edge_index, W1, b1, W2, b2):
    src = jnp.concatenate(
        [edge_index[0], jnp.zeros((EPAD,), jnp.int32)]
    ).reshape(NW, GB, KB, CHUNK)
    dst = jnp.concatenate(
        [edge_index[1], jnp.full((EPAD,), N, jnp.int32)]
    ).reshape(NW, GB, KB, CHUNK)

    degp = _sc_degree(dst)
    h1 = _tc_linear(input_feature, W1, b1)
    h1p = _tc_prescale(h1, degp)
    agg1 = _sc_gather_add(h1p, src, dst)
    h2p = _tc_mid(agg1, degp, W2, b2)
    agg2 = _sc_gather_add(h2p, src, dst)
    return _tc_logsoftmax(agg2, degp)


# speedup vs baseline: 2.4730x; 1.3742x over previous
import functools

import jax
import jax.numpy as jnp
from jax import lax
from jax.experimental import pallas as pl
from jax.experimental.pallas import tpu as pltpu
from jax.experimental.pallas import tpu_sc as plsc

N = 10000
D = 128
NC = 2
NS = 16
NW = NC * NS
CHUNK = 125
CHUNKS = 80
NE = 320000
NPAD = 10112
RPT = NPAD // NS

DEGW = 128
BLK = 1000
NBLK = N // BLK

def _mesh():
    return plsc.VectorSubcoreMesh(core_axis_name="c", subcore_axis_name="s")


def _zero_rows(zbuf, nrows, width):
    @pl.loop(0, nrows)
    def _(i):
        @pl.loop(0, width, step=16)
        def _(j):
            zbuf[i, pl.ds(j, 16)] = jnp.zeros((16,), jnp.float32)


def _zero_acc_slice(acc_sh, zbuf, sid):
    base = sid * RPT

    @pl.loop(0, RPT, step=8)
    def _(r):
        pltpu.sync_copy(zbuf, acc_sh.at[pl.ds(base + r, 8)])


def _sc_degree(dst_idx):
    @functools.partial(
        pl.kernel,
        out_type=jax.ShapeDtypeStruct((NC, NPAD, DEGW), jnp.float32),
        mesh=_mesh(),
        scratch_types=[
            pltpu.VMEM_SHARED((NPAD, DEGW), jnp.float32),
            pltpu.VMEM((CHUNKS, CHUNK), jnp.int32),
            pltpu.VMEM((CHUNK, DEGW), jnp.float32),
            pltpu.VMEM((8, DEGW), jnp.float32),
        ],
    )
    def k(dst_hbm, out_hbm, acc_sh, dstv, ones_v, zbuf):
        cid = lax.axis_index("c")
        sid = lax.axis_index("s")
        wid = sid * NC + cid
        _zero_rows(zbuf, 8, DEGW)

        @pl.loop(0, CHUNK)
        def _(i):
            @pl.loop(0, DEGW, step=16)
            def _(j):
                ones_v[i, pl.ds(j, 16)] = jnp.ones((16,), jnp.float32)

        _zero_acc_slice(acc_sh, zbuf, sid)
        pltpu.sync_copy(dst_hbm.at[wid], dstv)
        plsc.subcore_barrier()

        @pl.loop(0, CHUNKS)
        def _(j):
            pltpu.sync_copy(ones_v, acc_sh.at[dstv.at[j]], add=True)

        plsc.subcore_barrier()
        pltpu.sync_copy(
            acc_sh.at[pl.ds(sid * RPT, RPT)],
            out_hbm.at[cid, pl.ds(sid * RPT, RPT)],
        )

    return k(dst_idx)


def _sc_gather_add(h, src_idx, dst_idx):
    @functools.partial(
        pl.kernel,
        out_type=jax.ShapeDtypeStruct((NC, NPAD, D), jnp.float32),
        mesh=_mesh(),
        scratch_types=[
            pltpu.VMEM_SHARED((NPAD, D), jnp.float32),
            pltpu.VMEM((CHUNKS, CHUNK), jnp.int32),
            pltpu.VMEM((CHUNKS, CHUNK), jnp.int32),
            pltpu.VMEM((CHUNK, D), jnp.float32),
            pltpu.VMEM((8, D), jnp.float32),
            pltpu.SemaphoreType.DMA,
        ],
    )
    def k(h_hbm, src_hbm, dst_hbm, out_hbm, acc_sh, srcv, dstv, rows,
          zbuf, sem):
        cid = lax.axis_index("c")
        sid = lax.axis_index("s")
        wid = sid * NC + cid
        _zero_rows(zbuf, 8, D)

        _zero_acc_slice(acc_sh, zbuf, sid)
        pltpu.sync_copy(src_hbm.at[wid], srcv)
        pltpu.sync_copy(dst_hbm.at[wid], dstv)
        plsc.subcore_barrier()

        @pl.loop(0, CHUNKS)
        def _(j):
            pltpu.async_copy(h_hbm.at[srcv.at[j]], rows, sem).wait()
            pltpu.sync_copy(rows, acc_sh.at[dstv.at[j]], add=True)

        plsc.subcore_barrier()
        pltpu.sync_copy(
            acc_sh.at[pl.ds(sid * RPT, RPT)],
            out_hbm.at[cid, pl.ds(sid * RPT, RPT)],
        )

    return k(h, src_idx, dst_idx)


def _tc_linear(x, W, b):
    def body(x_ref, w_ref, b_ref, o_ref):
        o_ref[...] = (
            jnp.dot(x_ref[...], w_ref[...], preferred_element_type=jnp.float32)
            + b_ref[...]
        )

    return pl.pallas_call(
        body,
        grid=(NBLK,),
        in_specs=[
            pl.BlockSpec((BLK, D), lambda i: (i, 0)),
            pl.BlockSpec((D, D), lambda i: (0, 0)),
            pl.BlockSpec((1, D), lambda i: (0, 0)),
        ],
        out_specs=pl.BlockSpec((BLK, D), lambda i: (i, 0)),
        out_shape=jax.ShapeDtypeStruct((N, D), jnp.float32),
    )(x, W, b.reshape(1, D))


def _dinv_block(deg_ref):
    d = deg_ref[0, :, 0:1] + deg_ref[1, :, 0:1]
    return lax.rsqrt(jnp.maximum(d, 1.0))


def _tc_prescale(h, degp):
    def body(h_ref, deg_ref, o_ref):
        o_ref[...] = h_ref[...] * _dinv_block(deg_ref)

    return pl.pallas_call(
        body,
        grid=(NBLK,),
        in_specs=[
            pl.BlockSpec((BLK, D), lambda i: (i, 0)),
            pl.BlockSpec((NC, BLK, DEGW), lambda i: (0, i, 0)),
        ],
        out_specs=pl.BlockSpec((BLK, D), lambda i: (i, 0)),
        out_shape=jax.ShapeDtypeStruct((N, D), jnp.float32),
    )(h, degp)


def _tc_mid(aggp, degp, W, b):
    def body(agg_ref, deg_ref, w_ref, b_ref, o_ref):
        dinv = _dinv_block(deg_ref)
        x1 = jnp.maximum((agg_ref[0] + agg_ref[1]) * dinv, 0.0)
        h2 = jnp.dot(x1, w_ref[...], preferred_element_type=jnp.float32) + b_ref[...]
        o_ref[...] = h2 * dinv

    return pl.pallas_call(
        body,
        grid=(NBLK,),
        in_specs=[
            pl.BlockSpec((NC, BLK, D), lambda i: (0, i, 0)),
            pl.BlockSpec((NC, BLK, DEGW), lambda i: (0, i, 0)),
            pl.BlockSpec((D, D), lambda i: (0, 0)),
            pl.BlockSpec((1, D), lambda i: (0, 0)),
        ],
        out_specs=pl.BlockSpec((BLK, D), lambda i: (i, 0)),
        out_shape=jax.ShapeDtypeStruct((N, D), jnp.float32),
    )(aggp, degp, W, b.reshape(1, D))


def _tc_logsoftmax(aggp, degp):
    def body(agg_ref, deg_ref, o_ref):
        a = (agg_ref[0] + agg_ref[1]) * _dinv_block(deg_ref)
        m = jnp.max(a, axis=1, keepdims=True)
        ex = jnp.exp(a - m)
        lse = jnp.log(jnp.sum(ex, axis=1, keepdims=True)) + m
        o_ref[...] = a - lse

    return pl.pallas_call(
        body,
        grid=(NBLK,),
        in_specs=[
            pl.BlockSpec((NC, BLK, D), lambda i: (0, i, 0)),
            pl.BlockSpec((NC, BLK, DEGW), lambda i: (0, i, 0)),
        ],
        out_specs=pl.BlockSpec((BLK, D), lambda i: (i, 0)),
        out_shape=jax.ShapeDtypeStruct((N, D), jnp.float32),
    )(aggp, degp)


def kernel(input_feature, edge_index, W1, b1, W2, b2):
    src = edge_index[0].reshape(NW, CHUNKS, CHUNK)
    dst = edge_index[1].reshape(NW, CHUNKS, CHUNK)

    degp = _sc_degree(dst)
    h1 = _tc_linear(input_feature, W1, b1)
    h1p = _tc_prescale(h1, degp)
    agg1 = _sc_gather_add(h1p, src, dst)
    h2p = _tc_mid(agg1, degp, W2, b2)
    agg2 = _sc_gather_add(h2p, src, dst)
    return _tc_logsoftmax(agg2, degp)


# speedup vs baseline: 2.5226x; 1.0200x over previous
import functools

import jax
import jax.numpy as jnp
from jax import lax
from jax.experimental import pallas as pl
from jax.experimental.pallas import tpu as pltpu
from jax.experimental.pallas import tpu_sc as plsc

N = 10000
D = 128
NC = 2
NS = 16
NW = NC * NS
CHUNK = 125
CHUNKS = 80
NE = 320000
NPAD = 10112
RPT = NPAD // NS

DEGW = 128
BLK = 1000
NBLK = N // BLK

def _mesh():
    return plsc.VectorSubcoreMesh(core_axis_name="c", subcore_axis_name="s")


def _zero_rows(zbuf, nrows, width):
    @pl.loop(0, nrows)
    def _(i):
        @pl.loop(0, width, step=16)
        def _(j):
            zbuf[i, pl.ds(j, 16)] = jnp.zeros((16,), jnp.float32)


def _zero_acc_slice(acc_sh, zbuf, zbuf_tail, sid):
    base = sid * RPT

    @pl.loop(0, 600, step=40)
    def _(r):
        pltpu.sync_copy(zbuf, acc_sh.at[pl.ds(base + r, 40)])

    pltpu.sync_copy(zbuf_tail, acc_sh.at[pl.ds(base + 600, 32)])


def _sc_degree(dst_idx):
    @functools.partial(
        pl.kernel,
        out_type=jax.ShapeDtypeStruct((NC, NPAD, DEGW), jnp.float32),
        mesh=_mesh(),
        scratch_types=[
            pltpu.VMEM_SHARED((NPAD, DEGW), jnp.float32),
            pltpu.VMEM((CHUNKS, CHUNK), jnp.int32),
            pltpu.VMEM((CHUNK, DEGW), jnp.float32),
            pltpu.VMEM((40, DEGW), jnp.float32),
            pltpu.VMEM((32, DEGW), jnp.float32),
        ],
    )
    def k(dst_hbm, out_hbm, acc_sh, dstv, ones_v, zbuf, zbuf_tail):
        cid = lax.axis_index("c")
        sid = lax.axis_index("s")
        wid = sid * NC + cid
        _zero_rows(zbuf, 40, DEGW)
        _zero_rows(zbuf_tail, 32, DEGW)

        @pl.loop(0, CHUNK)
        def _(i):
            @pl.loop(0, DEGW, step=16)
            def _(j):
                ones_v[i, pl.ds(j, 16)] = jnp.ones((16,), jnp.float32)

        _zero_acc_slice(acc_sh, zbuf, zbuf_tail, sid)
        pltpu.sync_copy(dst_hbm.at[wid], dstv)
        plsc.subcore_barrier()

        @pl.loop(0, CHUNKS)
        def _(j):
            pltpu.sync_copy(ones_v, acc_sh.at[dstv.at[j]], add=True)

        plsc.subcore_barrier()
        pltpu.sync_copy(
            acc_sh.at[pl.ds(sid * RPT, RPT)],
            out_hbm.at[cid, pl.ds(sid * RPT, RPT)],
        )

    return k(dst_idx)


def _sc_gather_add(h, src_idx, dst_idx):
    @functools.partial(
        pl.kernel,
        out_type=jax.ShapeDtypeStruct((NC, NPAD, D), jnp.float32),
        mesh=_mesh(),
        scratch_types=[
            pltpu.VMEM_SHARED((NPAD, D), jnp.float32),
            pltpu.VMEM((CHUNKS, CHUNK), jnp.int32),
            pltpu.VMEM((CHUNKS, CHUNK), jnp.int32),
            pltpu.VMEM((CHUNK, D), jnp.float32),
            pltpu.VMEM((40, D), jnp.float32),
            pltpu.VMEM((32, D), jnp.float32),
            pltpu.SemaphoreType.DMA,
        ],
    )
    def k(h_hbm, src_hbm, dst_hbm, out_hbm, acc_sh, srcv, dstv, rows,
          zbuf, zbuf_tail, sem):
        cid = lax.axis_index("c")
        sid = lax.axis_index("s")
        wid = sid * NC + cid
        _zero_rows(zbuf, 40, D)
        _zero_rows(zbuf_tail, 32, D)
        _zero_acc_slice(acc_sh, zbuf, zbuf_tail, sid)
        pltpu.sync_copy(src_hbm.at[wid], srcv)
        pltpu.sync_copy(dst_hbm.at[wid], dstv)
        plsc.subcore_barrier()

        @pl.loop(0, CHUNKS)
        def _(j):
            pltpu.async_copy(h_hbm.at[srcv.at[j]], rows, sem).wait()
            pltpu.sync_copy(rows, acc_sh.at[dstv.at[j]], add=True)

        plsc.subcore_barrier()
        pltpu.sync_copy(
            acc_sh.at[pl.ds(sid * RPT, RPT)],
            out_hbm.at[cid, pl.ds(sid * RPT, RPT)],
        )

    return k(h, src_idx, dst_idx)


def _tc_linear(x, W, b):
    def body(x_ref, w_ref, b_ref, o_ref):
        o_ref[...] = (
            jnp.dot(x_ref[...], w_ref[...], preferred_element_type=jnp.float32)
            + b_ref[...]
        )

    return pl.pallas_call(
        body,
        grid=(NBLK,),
        in_specs=[
            pl.BlockSpec((BLK, D), lambda i: (i, 0)),
            pl.BlockSpec((D, D), lambda i: (0, 0)),
            pl.BlockSpec((1, D), lambda i: (0, 0)),
        ],
        out_specs=pl.BlockSpec((BLK, D), lambda i: (i, 0)),
        out_shape=jax.ShapeDtypeStruct((N, D), jnp.float32),
    )(x, W, b.reshape(1, D))


def _dinv_block(deg_ref):
    d = deg_ref[0, :, 0:1] + deg_ref[1, :, 0:1]
    return lax.rsqrt(jnp.maximum(d, 1.0))


def _tc_prescale(h, degp):
    def body(h_ref, deg_ref, o_ref):
        o_ref[...] = h_ref[...] * _dinv_block(deg_ref)

    return pl.pallas_call(
        body,
        grid=(NBLK,),
        in_specs=[
            pl.BlockSpec((BLK, D), lambda i: (i, 0)),
            pl.BlockSpec((NC, BLK, DEGW), lambda i: (0, i, 0)),
        ],
        out_specs=pl.BlockSpec((BLK, D), lambda i: (i, 0)),
        out_shape=jax.ShapeDtypeStruct((N, D), jnp.float32),
    )(h, degp)


def _tc_mid(aggp, degp, W, b):
    def body(agg_ref, deg_ref, w_ref, b_ref, o_ref):
        dinv = _dinv_block(deg_ref)
        x1 = jnp.maximum((agg_ref[0] + agg_ref[1]) * dinv, 0.0)
        h2 = jnp.dot(x1, w_ref[...], preferred_element_type=jnp.float32) + b_ref[...]
        o_ref[...] = h2 * dinv

    return pl.pallas_call(
        body,
        grid=(NBLK,),
        in_specs=[
            pl.BlockSpec((NC, BLK, D), lambda i: (0, i, 0)),
            pl.BlockSpec((NC, BLK, DEGW), lambda i: (0, i, 0)),
            pl.BlockSpec((D, D), lambda i: (0, 0)),
            pl.BlockSpec((1, D), lambda i: (0, 0)),
        ],
        out_specs=pl.BlockSpec((BLK, D), lambda i: (i, 0)),
        out_shape=jax.ShapeDtypeStruct((N, D), jnp.float32),
    )(aggp, degp, W, b.reshape(1, D))


def _tc_logsoftmax(aggp, degp):
    def body(agg_ref, deg_ref, o_ref):
        a = (agg_ref[0] + agg_ref[1]) * _dinv_block(deg_ref)
        m = jnp.max(a, axis=1, keepdims=True)
        ex = jnp.exp(a - m)
        lse = jnp.log(jnp.sum(ex, axis=1, keepdims=True)) + m
        o_ref[...] = a - lse

    return pl.pallas_call(
        body,
        grid=(NBLK,),
        in_specs=[
            pl.BlockSpec((NC, BLK, D), lambda i: (0, i, 0)),
            pl.BlockSpec((NC, BLK, DEGW), lambda i: (0, i, 0)),
        ],
        out_specs=pl.BlockSpec((BLK, D), lambda i: (i, 0)),
        out_shape=jax.ShapeDtypeStruct((N, D), jnp.float32),
    )(aggp, degp)


def kernel(input_feature, edge_index, W1, b1, W2, b2):
    src = edge_index[0].reshape(NW, CHUNKS, CHUNK)
    dst = edge_index[1].reshape(NW, CHUNKS, CHUNK)

    degp = _sc_degree(dst)
    h1 = _tc_linear(input_feature, W1, b1)
    h1p = _tc_prescale(h1, degp)
    agg1 = _sc_gather_add(h1p, src, dst)
    h2p = _tc_mid(agg1, degp, W2, b2)
    agg2 = _sc_gather_add(h2p, src, dst)
    return _tc_logsoftmax(agg2, degp)


# speedup vs baseline: 2.5544x; 1.0126x over previous
import functools

import jax
import jax.numpy as jnp
from jax import lax
from jax.experimental import pallas as pl
from jax.experimental.pallas import tpu as pltpu
from jax.experimental.pallas import tpu_sc as plsc

N = 10000
D = 128
NC = 2
NS = 16
NW = NC * NS
CHUNK = 125
CHUNKS = 80
NE = 320000
NPAD = 10112
RPT = NPAD // NS

DEGW = 128
BLK = 1000
NBLK = N // BLK

def _mesh():
    return plsc.VectorSubcoreMesh(core_axis_name="c", subcore_axis_name="s")


def _zero_rows(zbuf, nrows, width):
    @pl.loop(0, nrows)
    def _(i):
        @pl.loop(0, width, step=16)
        def _(j):
            zbuf[i, pl.ds(j, 16)] = jnp.zeros((16,), jnp.float32)


def _zero_acc_slice(acc_sh, zbuf, zbuf_tail, sid):
    base = sid * RPT

    @pl.loop(0, 600, step=40)
    def _(r):
        pltpu.sync_copy(zbuf, acc_sh.at[pl.ds(base + r, 40)])

    pltpu.sync_copy(zbuf_tail, acc_sh.at[pl.ds(base + 600, 32)])


def _sc_degree(dst_idx):
    @functools.partial(
        pl.kernel,
        out_type=jax.ShapeDtypeStruct((NC, NPAD, DEGW), jnp.float32),
        mesh=_mesh(),
        scratch_types=[
            pltpu.VMEM_SHARED((NPAD, DEGW), jnp.float32),
            pltpu.VMEM((CHUNKS, CHUNK), jnp.int32),
            pltpu.VMEM((CHUNK, DEGW), jnp.float32),
            pltpu.VMEM((40, DEGW), jnp.float32),
            pltpu.VMEM((32, DEGW), jnp.float32),
            pltpu.SemaphoreType.DMA,
        ],
    )
    def k(dst_hbm, out_hbm, acc_sh, dstv, ones_v, zbuf, zbuf_tail, sem):
        cid = lax.axis_index("c")
        sid = lax.axis_index("s")
        wid = sid * NC + cid
        pltpu.async_copy(dst_hbm.at[wid], dstv, sem)
        _zero_rows(zbuf, 40, DEGW)
        _zero_rows(zbuf_tail, 32, DEGW)

        @pl.loop(0, CHUNK)
        def _(i):
            @pl.loop(0, DEGW, step=16)
            def _(j):
                ones_v[i, pl.ds(j, 16)] = jnp.ones((16,), jnp.float32)

        _zero_acc_slice(acc_sh, zbuf, zbuf_tail, sid)
        pltpu.make_async_copy(dst_hbm.at[wid], dstv, sem).wait()
        plsc.subcore_barrier()

        @pl.loop(0, CHUNKS)
        def _(j):
            pltpu.sync_copy(ones_v, acc_sh.at[dstv.at[j]], add=True)

        plsc.subcore_barrier()
        pltpu.sync_copy(
            acc_sh.at[pl.ds(sid * RPT, RPT)],
            out_hbm.at[cid, pl.ds(sid * RPT, RPT)],
        )

    return k(dst_idx)


def _sc_gather_add(h, src_idx, dst_idx):
    @functools.partial(
        pl.kernel,
        out_type=jax.ShapeDtypeStruct((NC, NPAD, D), jnp.float32),
        mesh=_mesh(),
        scratch_types=[
            pltpu.VMEM_SHARED((NPAD, D), jnp.float32),
            pltpu.VMEM((CHUNKS, CHUNK), jnp.int32),
            pltpu.VMEM((CHUNKS, CHUNK), jnp.int32),
            pltpu.VMEM((CHUNK, D), jnp.float32),
            pltpu.VMEM((40, D), jnp.float32),
            pltpu.VMEM((32, D), jnp.float32),
            pltpu.SemaphoreType.DMA,
        ],
    )
    def k(h_hbm, src_hbm, dst_hbm, out_hbm, acc_sh, srcv, dstv, rows,
          zbuf, zbuf_tail, sem):
        cid = lax.axis_index("c")
        sid = lax.axis_index("s")
        wid = sid * NC + cid
        pltpu.async_copy(src_hbm.at[wid], srcv, sem)
        pltpu.async_copy(dst_hbm.at[wid], dstv, sem)
        _zero_rows(zbuf, 40, D)
        _zero_rows(zbuf_tail, 32, D)
        _zero_acc_slice(acc_sh, zbuf, zbuf_tail, sid)
        pltpu.make_async_copy(src_hbm.at[wid], srcv, sem).wait()
        pltpu.make_async_copy(dst_hbm.at[wid], dstv, sem).wait()
        plsc.subcore_barrier()

        @pl.loop(0, CHUNKS)
        def _(j):
            pltpu.async_copy(h_hbm.at[srcv.at[j]], rows, sem).wait()
            pltpu.sync_copy(rows, acc_sh.at[dstv.at[j]], add=True)

        plsc.subcore_barrier()
        pltpu.sync_copy(
            acc_sh.at[pl.ds(sid * RPT, RPT)],
            out_hbm.at[cid, pl.ds(sid * RPT, RPT)],
        )

    return k(h, src_idx, dst_idx)


def _tc_linear(x, W, b):
    def body(x_ref, w_ref, b_ref, o_ref):
        o_ref[...] = (
            jnp.dot(x_ref[...], w_ref[...], preferred_element_type=jnp.float32)
            + b_ref[...]
        )

    return pl.pallas_call(
        body,
        grid=(NBLK,),
        in_specs=[
            pl.BlockSpec((BLK, D), lambda i: (i, 0)),
            pl.BlockSpec((D, D), lambda i: (0, 0)),
            pl.BlockSpec((1, D), lambda i: (0, 0)),
        ],
        out_specs=pl.BlockSpec((BLK, D), lambda i: (i, 0)),
        out_shape=jax.ShapeDtypeStruct((N, D), jnp.float32),
    )(x, W, b.reshape(1, D))


def _dinv_block(deg_ref):
    d = deg_ref[0, :, 0:1] + deg_ref[1, :, 0:1]
    return lax.rsqrt(jnp.maximum(d, 1.0))


def _tc_prescale(h, degp):
    def body(h_ref, deg_ref, o_ref):
        o_ref[...] = h_ref[...] * _dinv_block(deg_ref)

    return pl.pallas_call(
        body,
        grid=(NBLK,),
        in_specs=[
            pl.BlockSpec((BLK, D), lambda i: (i, 0)),
            pl.BlockSpec((NC, BLK, DEGW), lambda i: (0, i, 0)),
        ],
        out_specs=pl.BlockSpec((BLK, D), lambda i: (i, 0)),
        out_shape=jax.ShapeDtypeStruct((N, D), jnp.float32),
    )(h, degp)


def _tc_mid(aggp, degp, W, b):
    def body(agg_ref, deg_ref, w_ref, b_ref, o_ref):
        dinv = _dinv_block(deg_ref)
        x1 = jnp.maximum((agg_ref[0] + agg_ref[1]) * dinv, 0.0)
        h2 = jnp.dot(x1, w_ref[...], preferred_element_type=jnp.float32) + b_ref[...]
        o_ref[...] = h2 * dinv

    return pl.pallas_call(
        body,
        grid=(NBLK,),
        in_specs=[
            pl.BlockSpec((NC, BLK, D), lambda i: (0, i, 0)),
            pl.BlockSpec((NC, BLK, DEGW), lambda i: (0, i, 0)),
            pl.BlockSpec((D, D), lambda i: (0, 0)),
            pl.BlockSpec((1, D), lambda i: (0, 0)),
        ],
        out_specs=pl.BlockSpec((BLK, D), lambda i: (i, 0)),
        out_shape=jax.ShapeDtypeStruct((N, D), jnp.float32),
    )(aggp, degp, W, b.reshape(1, D))


def _tc_logsoftmax(aggp, degp):
    def body(agg_ref, deg_ref, o_ref):
        a = (agg_ref[0] + agg_ref[1]) * _dinv_block(deg_ref)
        m = jnp.max(a, axis=1, keepdims=True)
        ex = jnp.exp(a - m)
        lse = jnp.log(jnp.sum(ex, axis=1, keepdims=True)) + m
        o_ref[...] = a - lse

    return pl.pallas_call(
        body,
        grid=(NBLK,),
        in_specs=[
            pl.BlockSpec((NC, BLK, D), lambda i: (0, i, 0)),
            pl.BlockSpec((NC, BLK, DEGW), lambda i: (0, i, 0)),
        ],
        out_specs=pl.BlockSpec((BLK, D), lambda i: (i, 0)),
        out_shape=jax.ShapeDtypeStruct((N, D), jnp.float32),
    )(aggp, degp)


def kernel(input_feature, edge_index, W1, b1, W2, b2):
    src = edge_index[0].reshape(NW, CHUNKS, CHUNK)
    dst = edge_index[1].reshape(NW, CHUNKS, CHUNK)

    degp = _sc_degree(dst)
    h1 = _tc_linear(input_feature, W1, b1)
    h1p = _tc_prescale(h1, degp)
    agg1 = _sc_gather_add(h1p, src, dst)
    h2p = _tc_mid(agg1, degp, W2, b2)
    agg2 = _sc_gather_add(h2p, src, dst)
    return _tc_logsoftmax(agg2, degp)
